# Initial kernel scaffold; baseline (speedup 1.0000x reference)
import jax
import jax.numpy as jnp
from jax.experimental import pallas as pl


def kernel(x, W1, b1, W2, b2, Wr, br):
    raise NotImplementedError("write your pallas kernel here")


# speedup vs baseline: 1.5944x; 1.5944x over previous
import jax
import jax.numpy as jnp
from jax import lax
from jax.experimental import pallas as pl

B, N_OBJ, D = 4, 2048, 768
E = 8
H = 768
O = 768
T = B * N_OBJ
TB = 256
NB = T // TB


def _dense_body(x_ref, W1_ref, b1_ref, W2_ref, b2_ref, Wr_ref, br_ref, out_ref):
    xb = x_ref[...]
    logits = lax.dot_general(xb, Wr_ref[...], (((1,), (1,)), ((), ())),
                             preferred_element_type=jnp.float32) + br_ref[...]
    ids = lax.broadcasted_iota(jnp.int32, (TB, E), 1)
    m0 = jnp.max(logits, axis=1, keepdims=True)
    a0 = jnp.min(jnp.where(logits == m0, ids, E), axis=1, keepdims=True)
    l1 = jnp.where(ids == a0, -jnp.inf, logits)
    m1 = jnp.max(l1, axis=1, keepdims=True)
    a1 = jnp.min(jnp.where(l1 == m1, ids, E), axis=1, keepdims=True)
    c0 = 1.0 / (1.0 + jnp.exp(m1 - m0))
    c1 = 1.0 - c0

    acc = jnp.zeros((TB, O), jnp.float32)
    for e in range(E):
        h = lax.dot_general(xb, W1_ref[e], (((1,), (1,)), ((), ())),
                            preferred_element_type=jnp.float32) + b1_ref[e]
        h = jnp.maximum(h, 0.0)
        y = lax.dot_general(h, W2_ref[e], (((1,), (1,)), ((), ())),
                            preferred_element_type=jnp.float32) + b2_ref[e]
        coef = jnp.where(a0 == e, c0, 0.0) + jnp.where(a1 == e, c1, 0.0)
        acc = acc + y * coef
    out_ref[...] = acc


def kernel(x, W1, b1, W2, b2, Wr, br):
    xf = x.reshape(T, D)
    b1r = b1.reshape(E, 1, H)
    b2r = b2.reshape(E, 1, O)
    brr = br.reshape(1, E)
    out = pl.pallas_call(
        _dense_body,
        grid=(NB,),
        in_specs=[
            pl.BlockSpec((TB, D), lambda i: (i, 0)),
            pl.BlockSpec((E, H, D), lambda i: (0, 0, 0)),
            pl.BlockSpec((E, 1, H), lambda i: (0, 0, 0)),
            pl.BlockSpec((E, O, H), lambda i: (0, 0, 0)),
            pl.BlockSpec((E, 1, O), lambda i: (0, 0, 0)),
            pl.BlockSpec((E, D), lambda i: (0, 0)),
            pl.BlockSpec((1, E), lambda i: (0, 0)),
        ],
        out_specs=pl.BlockSpec((TB, O), lambda i: (i, 0)),
        out_shape=jax.ShapeDtypeStruct((T, O), jnp.float32),
    )(xf, W1, b1r, W2, b2r, Wr, brr)
    return out.reshape(B, N_OBJ, O)
